# baseline (speedup 1.0000x reference)
import jax
import jax.numpy as jnp
from jax import lax
from jax.experimental import pallas as pl
from jax.experimental.pallas import tpu as pltpu
from jax.experimental.pallas import tpu_sc as plsc

N = 100000
E = 6400000
NC = 2
NS = 16
L = 16
W = NC * NS
C = 1280
NW = E // C
FULL_T = NW // W
EXTRA_W = NW % W
D = 3
GROUPS = FULL_T // D
ACC_N = 102400
SLICE = ACC_N // NS
UNROLL = 8
_PIECES = [(q * C, C) for q in range(SLICE // C)]


def _sc_edge_pass(x, edge_index, edge_weight):
    mesh = plsc.VectorSubcoreMesh(core_axis_name="c", subcore_axis_name="s")

    @pl.kernel(
        out_type=jax.ShapeDtypeStruct((NC * ACC_N,), jnp.float32),
        mesh=mesh,
        compiler_params=pltpu.CompilerParams(needs_layout_passes=False),
        scratch_types=[
            pltpu.VMEM((N,), jnp.float32),
            [pltpu.VMEM((2, C), jnp.int32) for _ in range(D)],
            [pltpu.VMEM((C,), jnp.int32) for _ in range(D)],
            [pltpu.VMEM((C,), jnp.float32) for _ in range(D)],
            [pltpu.VMEM((C,), jnp.float32) for _ in range(D)],
            [pltpu.SemaphoreType.DMA for _ in range(D)],
            [pltpu.SemaphoreType.DMA for _ in range(D)],
            pltpu.VMEM_SHARED((ACC_N,), jnp.float32),
        ],
    )
    def edge_pass(x_hbm, ei_hbm, w_hbm, out_hbm,
                  state_v, e_v, dst_v, w_v, msg_v, sem_in, sem_sc, acc_sh):
        cid = lax.axis_index("c")
        sid = lax.axis_index("s")
        wid = sid * NC + cid

        pltpu.sync_copy(x_hbm, state_v)

        def zero_body(i, _):
            msg_v[0][pl.ds(i * L, L)] = jnp.zeros((L,), jnp.float32)
            return 0
        lax.fori_loop(0, C // L, zero_body, 0)
        for o, ln in _PIECES:
            pltpu.sync_copy(msg_v[0].at[pl.ds(0, ln)],
                            acc_sh.at[pl.ds(sid * SLICE + o, ln)])
        plsc.subcore_barrier()

        def issue_in(b, t):
            off = (wid + W * t) * C
            return (
                pltpu.async_copy(ei_hbm.at[:, pl.ds(off, C)], e_v[b], sem_in[b]),
                pltpu.async_copy(w_hbm.at[pl.ds(off, C)], w_v[b], sem_in[b]),
            )

        def compute(b):
            def gather_body(j, _):
                for u in range(UNROLL):
                    sl = pl.ds((j * UNROLL + u) * L, L)
                    vals = plsc.load_gather(state_v, [e_v[b][0, sl]])
                    msg_v[b][sl] = w_v[b][sl] * vals
                    dst_v[b][sl] = e_v[b][1, sl]
                return 0
            lax.fori_loop(0, C // (L * UNROLL), gather_body, 0)

        def do_windows(ts):
            ins = [issue_in(b, t) for b, t in enumerate(ts)]
            scs = []
            for b, _ in enumerate(ts):
                for d in ins[b]:
                    d.wait()
                compute(b)
                scs.append(pltpu.async_copy(
                    msg_v[b], acc_sh.at[dst_v[b]], sem_sc[b], add=True))
            for s in scs:
                s.wait()

        def group_body(g, _):
            do_windows([g * D, g * D + 1, g * D + 2])
            return 0
        lax.fori_loop(0, GROUPS, group_body, 0)

        @pl.when(wid < EXTRA_W)
        def _():
            do_windows([FULL_T])

        plsc.subcore_barrier()
        for o, ln in _PIECES:
            pltpu.sync_copy(acc_sh.at[pl.ds(sid * SLICE + o, ln)],
                            msg_v[0].at[pl.ds(0, ln)])
            pltpu.sync_copy(msg_v[0].at[pl.ds(0, ln)],
                            out_hbm.at[pl.ds(cid * ACC_N + sid * SLICE + o, ln)])

    return edge_pass(x, edge_index, edge_weight)


def _tc_finish(x2d, p02d, p12d):
    def finish(x_ref, p0_ref, p1_ref, o_ref):
        o_ref[...] = jnp.tanh(x_ref[...] + p0_ref[...] + p1_ref[...])

    return pl.pallas_call(
        finish,
        out_shape=jax.ShapeDtypeStruct(x2d.shape, jnp.float32),
    )(x2d, p02d, p12d)


def kernel(region_inputs_flat, edge_index, edge_weight):
    x = region_inputs_flat
    partials = _sc_edge_pass(x, edge_index.astype(jnp.int32), edge_weight)
    x2d = x.reshape(8, N // 8)
    p0 = partials[:N].reshape(8, N // 8)
    p1 = partials[ACC_N:ACC_N + N].reshape(8, N // 8)
    return _tc_finish(x2d, p0, p1).reshape(N)


# speedup vs baseline: 1.7316x; 1.7316x over previous
import jax
import jax.numpy as jnp
from jax import lax
from jax.experimental import pallas as pl
from jax.experimental.pallas import tpu as pltpu
from jax.experimental.pallas import tpu_sc as plsc

N = 100000
E = 6400000
NC = 2
NS = 16
L = 16
W = NC * NS
C = 1280
NW = E // C
FULL_T = NW // W
EXTRA_W = NW % W
D = 3
GROUPS = FULL_T // D
ACC_N = 102400
SLICE = ACC_N // NS
UNROLL = 8
_PIECES = [(q * C, C) for q in range(SLICE // C)]


def _sc_edge_pass(x, edge_index, edge_weight):
    mesh = plsc.VectorSubcoreMesh(core_axis_name="c", subcore_axis_name="s")

    @pl.kernel(
        out_type=jax.ShapeDtypeStruct((NC * ACC_N,), jnp.float32),
        mesh=mesh,
        compiler_params=pltpu.CompilerParams(needs_layout_passes=False),
        scratch_types=[
            pltpu.VMEM((N,), jnp.float32),
            [pltpu.VMEM((2, C), jnp.int32) for _ in range(D)],
            [pltpu.VMEM((C,), jnp.int32) for _ in range(D)],
            [pltpu.VMEM((C,), jnp.float32) for _ in range(D)],
            [pltpu.VMEM((C,), jnp.float32) for _ in range(D)],
            [pltpu.SemaphoreType.DMA for _ in range(D)],
            [pltpu.SemaphoreType.DMA for _ in range(D)],
            pltpu.VMEM_SHARED((ACC_N,), jnp.float32),
        ],
    )
    def edge_pass(x_hbm, ei_hbm, w_hbm, out_hbm,
                  state_v, e_v, dst_v, w_v, msg_v, sem_in, sem_sc, acc_sh):
        cid = lax.axis_index("c")
        sid = lax.axis_index("s")
        wid = sid * NC + cid

        pltpu.sync_copy(x_hbm, state_v)

        def zero_body(i, _):
            msg_v[0][pl.ds(i * L, L)] = jnp.zeros((L,), jnp.float32)
            return 0
        lax.fori_loop(0, C // L, zero_body, 0)
        for o, ln in _PIECES:
            pltpu.sync_copy(msg_v[0].at[pl.ds(0, ln)],
                            acc_sh.at[pl.ds(sid * SLICE + o, ln)])
        plsc.subcore_barrier()

        def issue_in(b, t):
            off = (wid + W * t) * C
            return (
                pltpu.async_copy(ei_hbm.at[:, pl.ds(off, C)], e_v[b], sem_in[b]),
                pltpu.async_copy(w_hbm.at[pl.ds(off, C)], w_v[b], sem_in[b]),
            )

        def compute(b):
            @plsc.parallel_loop(0, C, step=L, unroll=UNROLL)
            def gather_body(i):
                sl = pl.ds(i, L)
                vals = plsc.load_gather(state_v, [e_v[b][0, sl]])
                msg_v[b][sl] = w_v[b][sl] * vals
                dst_v[b][sl] = e_v[b][1, sl]

        def do_windows(ts):
            ins = [issue_in(b, t) for b, t in enumerate(ts)]
            scs = []
            for b, _ in enumerate(ts):
                for d in ins[b]:
                    d.wait()
                compute(b)
                scs.append(pltpu.async_copy(
                    msg_v[b], acc_sh.at[dst_v[b]], sem_sc[b], add=True))
            for s in scs:
                s.wait()

        def group_body(g, _):
            do_windows([g * D, g * D + 1, g * D + 2])
            return 0
        lax.fori_loop(0, GROUPS, group_body, 0)

        @pl.when(wid < EXTRA_W)
        def _():
            do_windows([FULL_T])

        plsc.subcore_barrier()
        for o, ln in _PIECES:
            pltpu.sync_copy(acc_sh.at[pl.ds(sid * SLICE + o, ln)],
                            msg_v[0].at[pl.ds(0, ln)])
            pltpu.sync_copy(msg_v[0].at[pl.ds(0, ln)],
                            out_hbm.at[pl.ds(cid * ACC_N + sid * SLICE + o, ln)])

    return edge_pass(x, edge_index, edge_weight)


def _tc_finish(x2d, p02d, p12d):
    def finish(x_ref, p0_ref, p1_ref, o_ref):
        o_ref[...] = jnp.tanh(x_ref[...] + p0_ref[...] + p1_ref[...])

    return pl.pallas_call(
        finish,
        out_shape=jax.ShapeDtypeStruct(x2d.shape, jnp.float32),
    )(x2d, p02d, p12d)


def kernel(region_inputs_flat, edge_index, edge_weight):
    x = region_inputs_flat
    partials = _sc_edge_pass(x, edge_index.astype(jnp.int32), edge_weight)
    x2d = x.reshape(8, N // 8)
    p0 = partials[:N].reshape(8, N // 8)
    p1 = partials[ACC_N:ACC_N + N].reshape(8, N // 8)
    return _tc_finish(x2d, p0, p1).reshape(N)


# speedup vs baseline: 1.7852x; 1.0309x over previous
import jax
import jax.numpy as jnp
from jax import lax
from jax.experimental import pallas as pl
from jax.experimental.pallas import tpu as pltpu
from jax.experimental.pallas import tpu_sc as plsc

N = 100000
E = 6400000
NC = 2
NS = 16
L = 16
W = NC * NS
C = 1280
NW = E // C
FULL_T = NW // W
EXTRA_W = NW % W
D = 3
GROUPS = FULL_T // D
ACC_N = 102400
SLICE = ACC_N // NS
UNROLL = 8
_PIECES = [(q * C, C) for q in range(SLICE // C)]


def _sc_edge_pass(x, edge_index, edge_weight):
    mesh = plsc.VectorSubcoreMesh(core_axis_name="c", subcore_axis_name="s")

    @pl.kernel(
        out_type=jax.ShapeDtypeStruct((NC * ACC_N,), jnp.float32),
        mesh=mesh,
        compiler_params=pltpu.CompilerParams(needs_layout_passes=False),
        scratch_types=[
            pltpu.VMEM((N,), jnp.float32),
            [pltpu.VMEM((2, C), jnp.int32) for _ in range(D)],
            [pltpu.VMEM((C,), jnp.int32) for _ in range(D)],
            [pltpu.VMEM((C,), jnp.float32) for _ in range(D)],
            [pltpu.VMEM((C,), jnp.float32) for _ in range(D)],
            [pltpu.SemaphoreType.DMA for _ in range(D)],
            [pltpu.SemaphoreType.DMA for _ in range(D)],
            pltpu.VMEM_SHARED((ACC_N,), jnp.float32),
        ],
    )
    def edge_pass(x_hbm, ei_hbm, w_hbm, out_hbm,
                  state_v, e_v, dst_v, w_v, msg_v, sem_in, sem_sc, acc_sh):
        cid = lax.axis_index("c")
        sid = lax.axis_index("s")
        wid = sid * NC + cid

        pltpu.sync_copy(x_hbm, state_v)

        def zero_body(i, _):
            msg_v[0][pl.ds(i * L, L)] = jnp.zeros((L,), jnp.float32)
            return 0
        lax.fori_loop(0, C // L, zero_body, 0)
        for o, ln in _PIECES:
            pltpu.sync_copy(msg_v[0].at[pl.ds(0, ln)],
                            acc_sh.at[pl.ds(sid * SLICE + o, ln)])
        plsc.subcore_barrier()

        def issue_in(b, t):
            off = (wid + W * t) * C
            return (
                pltpu.async_copy(ei_hbm.at[:, pl.ds(off, C)], e_v[b], sem_in[b]),
                pltpu.async_copy(w_hbm.at[pl.ds(off, C)], w_v[b], sem_in[b]),
            )

        def compute(b):
            @plsc.parallel_loop(0, C, step=L, unroll=UNROLL)
            def gather_body(i):
                sl = pl.ds(i, L)
                vals = plsc.load_gather(state_v, [e_v[b][0, sl]])
                msg_v[b][sl] = w_v[b][sl] * vals
                dst_v[b][sl] = e_v[b][1, sl]

        def do_windows(ts):
            ins = [issue_in(b, t) for b, t in enumerate(ts)]
            scs = []
            for b, _ in enumerate(ts):
                for d in ins[b]:
                    d.wait()
                compute(b)
                scs.append(pltpu.async_copy(
                    msg_v[b], acc_sh.at[dst_v[b]], sem_sc[b], add=True))
            for s in scs:
                s.wait()

        def group_body(g, _):
            do_windows([g * D, g * D + 1, g * D + 2])
            return 0
        lax.fori_loop(0, GROUPS, group_body, 0)

        @pl.when(wid < EXTRA_W)
        def _():
            do_windows([FULL_T])

        plsc.subcore_barrier()
        for o, ln in _PIECES:
            pltpu.sync_copy(acc_sh.at[pl.ds(sid * SLICE + o, ln)],
                            msg_v[0].at[pl.ds(0, ln)])
            pltpu.sync_copy(msg_v[0].at[pl.ds(0, ln)],
                            out_hbm.at[pl.ds(cid * ACC_N + sid * SLICE + o, ln)])

    return edge_pass(x, edge_index, edge_weight)


def _tc_finish(x, partials):
    def finish(x_ref, p_ref, o_ref):
        p0 = p_ref[pl.ds(0, N)]
        p1 = p_ref[pl.ds(ACC_N, N)]
        o_ref[...] = jnp.tanh(x_ref[...] + p0 + p1)

    return pl.pallas_call(
        finish,
        out_shape=jax.ShapeDtypeStruct((N,), jnp.float32),
    )(x, partials)


def kernel(region_inputs_flat, edge_index, edge_weight):
    x = region_inputs_flat
    partials = _sc_edge_pass(x, edge_index.astype(jnp.int32), edge_weight)
    return _tc_finish(x, partials)


# speedup vs baseline: 2.6187x; 1.4669x over previous
import jax
import jax.numpy as jnp
from jax import lax
from jax.experimental import pallas as pl
from jax.experimental.pallas import tpu as pltpu
from jax.experimental.pallas import tpu_sc as plsc

N = 100000
E = 6400000
NC = 2
NS = 16
L = 16
W = NC * NS
C = 1280
NW = E // C
FULL_T = NW // W
EXTRA_W = NW % W
D = 3
GROUPS = FULL_T // D
ACC_N = 102400
SLICE = ACC_N // NS
UNROLL = 8
_PIECES = [(q * C, C) for q in range(SLICE // C)]
ST_P = 5
ST_L = N // ST_P


def _sc_edge_pass(x, edge_index, edge_weight):
    mesh = plsc.VectorSubcoreMesh(core_axis_name="c", subcore_axis_name="s")

    @pl.kernel(
        out_type=jax.ShapeDtypeStruct((NC * ACC_N,), jnp.float32),
        mesh=mesh,
        compiler_params=pltpu.CompilerParams(needs_layout_passes=False),
        scratch_types=[
            pltpu.VMEM((N,), jnp.float32),
            [pltpu.VMEM((2, C), jnp.int32) for _ in range(D)],
            [pltpu.VMEM((C,), jnp.int32) for _ in range(D)],
            [pltpu.VMEM((C,), jnp.float32) for _ in range(D)],
            [pltpu.VMEM((C,), jnp.float32) for _ in range(D)],
            [pltpu.SemaphoreType.DMA for _ in range(D)],
            [pltpu.SemaphoreType.DMA for _ in range(D)],
            pltpu.VMEM_SHARED((ACC_N,), jnp.float32),
        ],
    )
    def edge_pass(x_hbm, ei_hbm, w_hbm, out_hbm,
                  state_v, e_v, dst_v, w_v, msg_v, sem_in, sem_sc, acc_sh):
        cid = lax.axis_index("c")
        sid = lax.axis_index("s")
        wid = sid * NC + cid

        def issue_in(b, t):
            off = (wid + W * t) * C
            return (
                pltpu.async_copy(ei_hbm.at[:, pl.ds(off, C)], e_v[b], sem_in[b]),
                pltpu.async_copy(w_hbm.at[pl.ds(off, C)], w_v[b], sem_in[b]),
            )

        def wait_in(b):
            pltpu.make_async_copy(ei_hbm.at[:, pl.ds(0, C)], e_v[b], sem_in[b]).wait()
            pltpu.make_async_copy(w_hbm.at[pl.ds(0, C)], w_v[b], sem_in[b]).wait()

        for b in range(D):
            issue_in(b, b)

        rot = lax.rem(wid, ST_P)
        for p in range(ST_P):
            o = lax.rem(jnp.int32(p) + rot, ST_P) * ST_L
            pltpu.sync_copy(x_hbm.at[pl.ds(o, ST_L)],
                            state_v.at[pl.ds(o, ST_L)])

        def zero_body(i, _):
            msg_v[0][pl.ds(i * L, L)] = jnp.zeros((L,), jnp.float32)
            return 0
        lax.fori_loop(0, C // L, zero_body, 0)
        for o, ln in _PIECES:
            pltpu.sync_copy(msg_v[0].at[pl.ds(0, ln)],
                            acc_sh.at[pl.ds(sid * SLICE + o, ln)])
        plsc.subcore_barrier()

        def compute(b):
            @plsc.parallel_loop(0, C, step=L, unroll=UNROLL)
            def gather_body(i):
                sl = pl.ds(i, L)
                vals = plsc.load_gather(state_v, [e_v[b][0, sl]])
                msg_v[b][sl] = w_v[b][sl] * vals
                dst_v[b][sl] = e_v[b][1, sl]

        def group_body(g, _):
            scs = []
            for b in range(D):
                wait_in(b)
                compute(b)

                @pl.when(g + 1 < GROUPS)
                def _():
                    issue_in(b, (g + 1) * D + b)
                scs.append(pltpu.async_copy(
                    msg_v[b], acc_sh.at[dst_v[b]], sem_sc[b], add=True))
            for s in scs:
                s.wait()
            return 0
        lax.fori_loop(0, GROUPS, group_body, 0)

        @pl.when(wid < EXTRA_W)
        def _():
            ins = issue_in(0, FULL_T)
            for d in ins:
                d.wait()
            compute(0)
            pltpu.async_copy(
                msg_v[0], acc_sh.at[dst_v[0]], sem_sc[0], add=True).wait()

        plsc.subcore_barrier()
        for o, ln in _PIECES:
            pltpu.sync_copy(acc_sh.at[pl.ds(sid * SLICE + o, ln)],
                            msg_v[0].at[pl.ds(0, ln)])
            pltpu.sync_copy(msg_v[0].at[pl.ds(0, ln)],
                            out_hbm.at[pl.ds(cid * ACC_N + sid * SLICE + o, ln)])

    return edge_pass(x, edge_index, edge_weight)


def _tc_finish(x, partials):
    def finish(x_ref, p_ref, o_ref):
        p0 = p_ref[pl.ds(0, N)]
        p1 = p_ref[pl.ds(ACC_N, N)]
        o_ref[...] = jnp.tanh(x_ref[...] + p0 + p1)

    return pl.pallas_call(
        finish,
        out_shape=jax.ShapeDtypeStruct((N,), jnp.float32),
    )(x, partials)


def kernel(region_inputs_flat, edge_index, edge_weight):
    x = region_inputs_flat
    partials = _sc_edge_pass(x, edge_index.astype(jnp.int32), edge_weight)
    return _tc_finish(x, partials)


# speedup vs baseline: 3.0004x; 1.1458x over previous
import jax
import jax.numpy as jnp
from jax import lax
from jax.experimental import pallas as pl
from jax.experimental.pallas import tpu as pltpu
from jax.experimental.pallas import tpu_sc as plsc

N = 100000
E = 6400000
NC = 2
NS = 16
L = 16
W = NC * NS
C = 1280
NW = E // C
FULL_T = NW // W
EXTRA_W = NW % W
D = 3
GROUPS = FULL_T // D
ACC_N = 102400
SLICE = ACC_N // NS
UNROLL = 8
_PIECES = [(q * C, C) for q in range(SLICE // C)]
ST_P = 5
ST_L = N // ST_P


def _sc_edge_pass(x, edge_index, edge_weight):
    mesh = plsc.VectorSubcoreMesh(core_axis_name="c", subcore_axis_name="s")

    @pl.kernel(
        out_type=jax.ShapeDtypeStruct((NC * ACC_N,), jnp.float32),
        mesh=mesh,
        compiler_params=pltpu.CompilerParams(needs_layout_passes=False),
        scratch_types=[
            pltpu.VMEM((N,), jnp.float32),
            [pltpu.VMEM((2, C), jnp.int32) for _ in range(D)],
            [pltpu.VMEM((C,), jnp.int32) for _ in range(D)],
            [pltpu.VMEM((C,), jnp.float32) for _ in range(D)],
            [pltpu.VMEM((C,), jnp.float32) for _ in range(D)],
            [pltpu.SemaphoreType.DMA for _ in range(D)],
            [pltpu.SemaphoreType.DMA for _ in range(D)],
            pltpu.VMEM_SHARED((ACC_N,), jnp.float32),
        ],
    )
    def edge_pass(x_hbm, ei_hbm, w_hbm, out_hbm,
                  state_v, e_v, dst_v, w_v, msg_v, sem_in, sem_sc, acc_sh):
        cid = lax.axis_index("c")
        sid = lax.axis_index("s")
        wid = sid * NC + cid

        def issue_in(b, t):
            off = (wid + W * t) * C
            return (
                pltpu.async_copy(ei_hbm.at[:, pl.ds(off, C)], e_v[b], sem_in[b]),
                pltpu.async_copy(w_hbm.at[pl.ds(off, C)], w_v[b], sem_in[b]),
            )

        def wait_in(b):
            pltpu.make_async_copy(ei_hbm.at[:, pl.ds(0, C)], e_v[b], sem_in[b]).wait()
            pltpu.make_async_copy(w_hbm.at[pl.ds(0, C)], w_v[b], sem_in[b]).wait()

        for b in range(D):
            issue_in(b, b)

        rot = lax.rem(wid, ST_P)
        for p in range(ST_P):
            o = lax.rem(jnp.int32(p) + rot, ST_P) * ST_L
            pltpu.sync_copy(x_hbm.at[pl.ds(o, ST_L)],
                            state_v.at[pl.ds(o, ST_L)])

        def zero_body(i, _):
            msg_v[0][pl.ds(i * L, L)] = jnp.zeros((L,), jnp.float32)
            return 0
        lax.fori_loop(0, C // L, zero_body, 0)
        for o, ln in _PIECES:
            pltpu.sync_copy(msg_v[0].at[pl.ds(0, ln)],
                            acc_sh.at[pl.ds(sid * SLICE + o, ln)])
        plsc.subcore_barrier()

        def compute(b):
            @plsc.parallel_loop(0, C, step=L, unroll=UNROLL)
            def gather_body(i):
                sl = pl.ds(i, L)
                vals = plsc.load_gather(state_v, [e_v[b][0, sl]])
                msg_v[b][sl] = w_v[b][sl] * vals
                dst_v[b][sl] = e_v[b][1, sl]

        def wait_sc(b):
            pltpu.make_async_copy(msg_v[b], acc_sh.at[dst_v[b]], sem_sc[b]).wait()

        def group_body(g, _):
            for b in range(D):
                wait_in(b)

                @pl.when(g > 0)
                def _():
                    wait_sc(b)
                compute(b)

                @pl.when(g + 1 < GROUPS)
                def _():
                    issue_in(b, (g + 1) * D + b)
                pltpu.async_copy(
                    msg_v[b], acc_sh.at[dst_v[b]], sem_sc[b], add=True)
            return 0
        lax.fori_loop(0, GROUPS, group_body, 0)
        for b in range(D):
            wait_sc(b)

        @pl.when(wid < EXTRA_W)
        def _():
            ins = issue_in(0, FULL_T)
            for d in ins:
                d.wait()
            compute(0)
            pltpu.async_copy(
                msg_v[0], acc_sh.at[dst_v[0]], sem_sc[0], add=True)
            wait_sc(0)

        plsc.subcore_barrier()
        for o, ln in _PIECES:
            pltpu.sync_copy(acc_sh.at[pl.ds(sid * SLICE + o, ln)],
                            msg_v[0].at[pl.ds(0, ln)])
            pltpu.sync_copy(msg_v[0].at[pl.ds(0, ln)],
                            out_hbm.at[pl.ds(cid * ACC_N + sid * SLICE + o, ln)])

    return edge_pass(x, edge_index, edge_weight)


def _tc_finish(x, partials):
    def finish(x_ref, p_ref, o_ref):
        p0 = p_ref[pl.ds(0, N)]
        p1 = p_ref[pl.ds(ACC_N, N)]
        o_ref[...] = jnp.tanh(x_ref[...] + p0 + p1)

    return pl.pallas_call(
        finish,
        out_shape=jax.ShapeDtypeStruct((N,), jnp.float32),
    )(x, partials)


def kernel(region_inputs_flat, edge_index, edge_weight):
    x = region_inputs_flat
    partials = _sc_edge_pass(x, edge_index.astype(jnp.int32), edge_weight)
    return _tc_finish(x, partials)
